# baseline (speedup 1.0000x reference)
import functools

import jax
import jax.numpy as jnp
from jax import lax
from jax.experimental import pallas as pl
from jax.experimental.pallas import tpu as pltpu
from jax.experimental.pallas import tpu_sc as plsc

_D = 32
_CHUNK = 1600


@functools.partial(jax.jit, static_argnums=(3, 4))
def _gather_weight(table, idx, w, n_rows, n_workers):
    rows_per_w = n_rows // n_workers
    n_chunks = rows_per_w // _CHUNK
    mesh = plsc.VectorSubcoreMesh(core_axis_name="c", subcore_axis_name="s")

    @functools.partial(
        pl.kernel,
        mesh=mesh,
        out_type=jax.ShapeDtypeStruct((n_rows, _D), jnp.float32),
        compiler_params=pltpu.CompilerParams(use_tc_tiling_on_sc=False),
        scratch_types=[
            pltpu.VMEM((_CHUNK,), jnp.int32),
            pltpu.VMEM((_CHUNK,), jnp.float32),
            pltpu.VMEM((_CHUNK, _D), jnp.float32),
            pltpu.SemaphoreType.DMA,
        ],
    )
    def k(table_hbm, idx_hbm, w_hbm, out_hbm, idx_v, w_v, rows_v, sem):
        nc = 2
        wid = lax.axis_index("s") * nc + lax.axis_index("c")
        base_w = wid * rows_per_w

        def chunk_body(g, carry):
            base = base_w + g * _CHUNK
            pltpu.sync_copy(idx_hbm.at[pl.ds(base, _CHUNK)], idx_v)
            gather = pltpu.async_copy(table_hbm.at[idx_v], rows_v, sem)
            pltpu.sync_copy(w_hbm.at[pl.ds(base, _CHUNK)], w_v)
            gather.wait()

            def group_body(g16, c):
                base16 = g16 * 16
                wvec = w_v[pl.ds(base16, 16)]
                for j in range(16):
                    wb = lax.broadcast(wvec[j], (16,))
                    i = base16 + j
                    rows_v[i, 0:16] = rows_v[i, 0:16] * wb
                    rows_v[i, 16:32] = rows_v[i, 16:32] * wb
                return c

            lax.fori_loop(0, _CHUNK // 16, group_body, 0)
            pltpu.sync_copy(rows_v, out_hbm.at[pl.ds(base, _CHUNK)])
            return carry

        lax.fori_loop(0, n_chunks, chunk_body, 0)

    return k(table, idx, w)


def kernel(target_indices, target_weights, embedding_weight):
    b, l = target_indices.shape
    n_rows = b * l
    idx = target_indices.reshape(n_rows).astype(jnp.int32)
    w = target_weights.reshape(n_rows)
    out = _gather_weight(embedding_weight, idx, w, n_rows, 32)
    return out.reshape(b, l, _D)
